# Initial kernel scaffold; baseline (speedup 1.0000x reference)
import jax
import jax.numpy as jnp
from jax.experimental import pallas as pl


def kernel(x, edge_index, batch, params):
    raise NotImplementedError("write your pallas kernel here")


# speedup vs baseline: 1.8643x; 1.8643x over previous
import functools

import jax
import jax.numpy as jnp
from jax import lax
from jax.experimental import pallas as pl
from jax.experimental.pallas import tpu as pltpu
from jax.experimental.pallas import tpu_sc as plsc

_N = 10000
_E = 320000
_H = 128
_G = 64
_NW = 32
_CH = 80
_EPW = _E // _NW


def _sc_gather_body(a_hbm, b_hbm, dst_hbm, src_hbm, t_hbm,
                    idxd, idxs, ra, rb, sem):
  wid = lax.axis_index("s") * 2 + lax.axis_index("c")
  base0 = wid * _EPW

  def chunk(k, carry):
    base = base0 + k * _CH
    pltpu.sync_copy(dst_hbm.at[pl.ds(base, _CH)], idxd)
    pltpu.sync_copy(src_hbm.at[pl.ds(base, _CH)], idxs)
    cpa = pltpu.async_copy(a_hbm.at[idxd], ra, sem)
    cpb = pltpu.async_copy(b_hbm.at[idxs], rb, sem)
    cpa.wait()
    cpb.wait()

    def row(r, c2):
      for f in range(_H // 16):
        sl = pl.ds(f * 16, 16)
        ra[r, sl] = jnp.maximum(ra[r, sl] + rb[r, sl], 0.0)
      return c2
    lax.fori_loop(0, _CH, row, 0, unroll=False)
    pltpu.sync_copy(ra, t_hbm.at[pl.ds(base, _CH)])
    return carry

  lax.fori_loop(0, _EPW // _CH, chunk, 0, unroll=False)


def _sc_gather(a, b, dst, src):
  mesh = plsc.VectorSubcoreMesh(core_axis_name="c", subcore_axis_name="s")
  fn = functools.partial(
      pl.kernel,
      mesh=mesh,
      out_type=jax.ShapeDtypeStruct((_E, _H), jnp.float32),
      scratch_types=[
          pltpu.VMEM((_CH,), jnp.int32),
          pltpu.VMEM((_CH,), jnp.int32),
          pltpu.VMEM((_CH, _H), jnp.float32),
          pltpu.VMEM((_CH, _H), jnp.float32),
          pltpu.SemaphoreType.DMA,
      ],
  )(_sc_gather_body)
  return fn(a, b, dst, src)


def _proj1_body(x_ref, wd_ref, wb_ref, cb_ref, a_ref, b_ref):
  x = x_ref[...]
  a_ref[...] = jnp.dot(x, wd_ref[...],
                       preferred_element_type=jnp.float32) + cb_ref[...]
  b_ref[...] = jnp.dot(x, wb_ref[...], preferred_element_type=jnp.float32)


def _proj1(x, wd, wb, cb):
  return pl.pallas_call(
      _proj1_body,
      out_shape=(jax.ShapeDtypeStruct((_N, _H), jnp.float32),
                 jax.ShapeDtypeStruct((_N, _H), jnp.float32)),
  )(x, wd, wb, cb)


def _projn_body(acc_ref, b2_ref, sn_ref, cn_ref, wd_ref, wb_ref, cb_ref,
                h_ref, a_ref, b_ref):
  acc = acc_ref[...]
  fixed = jnp.where(acc == -jnp.inf, 0.0, acc + b2_ref[...])
  h = jnp.maximum(fixed * sn_ref[...] + cn_ref[...], 0.0)
  h_ref[...] = h
  a_ref[...] = jnp.dot(h, wd_ref[...],
                       preferred_element_type=jnp.float32) + cb_ref[...]
  b_ref[...] = jnp.dot(h, wb_ref[...], preferred_element_type=jnp.float32)


def _projn(acc, b2, sn, cn, wd, wb, cb):
  return pl.pallas_call(
      _projn_body,
      out_shape=(jax.ShapeDtypeStruct((_N, _H), jnp.float32),
                 jax.ShapeDtypeStruct((_N, _H), jnp.float32),
                 jax.ShapeDtypeStruct((_N, _H), jnp.float32)),
  )(acc, b2, sn, cn, wd, wb, cb)


_EBLK = 6400

def _emm_body(t_ref, w_ref, u_ref):
  u_ref[...] = jnp.dot(t_ref[...], w_ref[...],
                       preferred_element_type=jnp.float32)


def _edge_matmul(t, w2):
  return pl.pallas_call(
      _emm_body,
      grid=(_E // _EBLK,),
      in_specs=[pl.BlockSpec((_EBLK, _H), lambda i: (i, 0)),
                pl.BlockSpec((_H, _H), lambda i: (0, 0))],
      out_specs=pl.BlockSpec((_EBLK, _H), lambda i: (i, 0)),
      out_shape=jax.ShapeDtypeStruct((_E, _H), jnp.float32),
  )(t, w2)


def _head_body(h1_ref, h2_ref, acc3_ref, b2_ref, sn_ref, cn_ref,
               lw_ref, lb_ref, batch_ref, c1_ref, c1b_ref, c2_ref, c2b_ref,
               out_ref):
  acc3 = acc3_ref[...]
  fixed = jnp.where(acc3 == -jnp.inf, 0.0, acc3 + b2_ref[...])
  h3 = jnp.maximum(fixed * sn_ref[...] + cn_ref[...], 0.0)
  hcat = jnp.concatenate([h1_ref[...], h2_ref[...], h3], axis=-1)
  h = jnp.dot(hcat, lw_ref[...],
              preferred_element_type=jnp.float32) + lb_ref[...]
  onehot = (batch_ref[...] ==
            lax.broadcasted_iota(jnp.int32, (1, _G), 1)).astype(jnp.float32)
  g = lax.dot_general(onehot, h, (((0,), (0,)), ((), ())),
                      preferred_element_type=jnp.float32)
  g = jnp.dot(g, c1_ref[...], preferred_element_type=jnp.float32) + c1b_ref[...]
  g = jnp.maximum(g, 0.0)
  out_ref[...] = jnp.dot(g, c2_ref[...],
                         preferred_element_type=jnp.float32) + c2b_ref[...]


def _head(h1, h2, acc3, b2, sn, cn, lw, lb, batch2d, c1, c1b, c2, c2b):
  return pl.pallas_call(
      _head_body,
      out_shape=jax.ShapeDtypeStruct((_G, _H), jnp.float32),
  )(h1, h2, acc3, b2, sn, cn, lw, lb, batch2d, c1, c1b, c2, c2b)


def _fold(params):
  eps = 1e-5
  folded = []
  for conv_p, norm_p in zip(params["convs"], params["norms"]):
    bn = conv_p["bn"]
    s = bn["gamma"] / jnp.sqrt(bn["var"] + eps)
    c = bn["beta"] - bn["mean"] * s
    w1 = conv_p["W1"]
    wtop, wbot = w1[:_H], w1[_H:]
    wd = (wtop - wbot) * s[None, :]
    wb = wbot * s[None, :]
    cb = (conv_p["b1"] * s + c)[None, :]
    sn = norm_p["gamma"] / jnp.sqrt(norm_p["var"] + eps)
    cn = (norm_p["beta"] - norm_p["mean"] * sn)[None, :]
    folded.append(dict(wd=wd, wb=wb, cb=cb, w2=conv_p["W2"],
                       b2=conv_p["b2"][None, :], sn=sn[None, :], cn=cn))
  cls = params["cls"]
  bn = cls["bn"]
  s = bn["gamma"] / jnp.sqrt(bn["var"] + eps)
  c = bn["beta"] - bn["mean"] * s
  c1 = cls["W1"] * s[None, :]
  c1b = (cls["b1"] * s + c)[None, :]
  c2 = jnp.zeros((2 * _H, _H), jnp.float32).at[:, :2].set(cls["W2"])
  c2b = jnp.zeros((1, _H), jnp.float32).at[0, :2].set(cls["b2"])
  return folded, cls, c1, c1b, c2, c2b


def kernel(x, edge_index, batch, params):
  src, dst = edge_index[0], edge_index[1]
  folded, cls, c1, c1b, c2, c2b = _fold(params)
  lin = params["lin"]

  f0 = folded[0]
  a, b = _proj1(x, f0["wd"], f0["wb"], f0["cb"])
  hs = []
  acc = None
  for li in range(3):
    f = folded[li]
    if li > 0:
      fp = folded[li - 1]
      h_prev, a, b = _projn(acc, fp["b2"], fp["sn"], fp["cn"],
                            f["wd"], f["wb"], f["cb"])
      hs.append(h_prev)
    t = _sc_gather(a, b, dst, src)
    u = _edge_matmul(t, f["w2"])
    acc = jax.ops.segment_max(u, dst, num_segments=_N)

  f2 = folded[2]
  out = _head(hs[0], hs[1], acc, f2["b2"], f2["sn"], f2["cn"],
              lin["W"], lin["b"][None, :], batch[:, None],
              c1, c1b, c2, c2b)
  return out[:, :2]
